# Initial kernel scaffold; baseline (speedup 1.0000x reference)
import jax
import jax.numpy as jnp
from jax.experimental import pallas as pl


def kernel(input_ids, table):
    raise NotImplementedError("write your pallas kernel here")


# speedup vs baseline: 5.3582x; 5.3582x over previous
import functools

import jax
import jax.numpy as jnp
from jax import lax
from jax.experimental import pallas as pl
from jax.experimental.pallas import tpu as pltpu
from jax.experimental.pallas import tpu_sc as plsc

D = 128

_info = plsc.get_sparse_core_info()
NC, NS = _info.num_cores, _info.num_subcores
NW = NC * NS

CHUNK = 128


def _threshold_body(t_ref, o_ref):
    o_ref[...] = (t_ref[...] >= 0).astype(jnp.float32)


def _sign_table(table):
    V, d = table.shape
    blk = 800
    return pl.pallas_call(
        _threshold_body,
        grid=(V // blk,),
        in_specs=[pl.BlockSpec((blk, d), lambda i: (i, 0))],
        out_specs=pl.BlockSpec((blk, d), lambda i: (i, 0)),
        out_shape=jax.ShapeDtypeStruct((V, d), jnp.float32),
    )(table)


def _make_gather(n_rows):
    per_w = n_rows // NW
    n_chunks = per_w // CHUNK
    mesh = plsc.VectorSubcoreMesh(core_axis_name="c", subcore_axis_name="s")

    @functools.partial(
        pl.kernel,
        mesh=mesh,
        out_type=jax.ShapeDtypeStruct((n_rows // CHUNK, CHUNK, D), jnp.float32),
        scratch_types=[
            pltpu.VMEM((n_chunks, CHUNK), jnp.int32),
            pltpu.VMEM((CHUNK, D), jnp.float32),
            pltpu.SemaphoreType.DMA,
        ],
    )
    def gather_k(ids_hbm, table_hbm, out_hbm, idx_v, rows_v, sem):
        wid = lax.axis_index("s") * NC + lax.axis_index("c")
        base = wid * n_chunks
        pltpu.sync_copy(ids_hbm.at[pl.ds(base, n_chunks)], idx_v)

        def body(j, carry):
            pltpu.async_copy(table_hbm.at[idx_v.at[j]], rows_v, sem).wait()
            pltpu.sync_copy(rows_v, out_hbm.at[base + j])
            return carry

        lax.fori_loop(0, n_chunks, body, 0)

    return gather_k


def kernel(input_ids, table):
    B, H = input_ids.shape
    n_rows = B * H
    signs = _sign_table(table)
    ids = input_ids.reshape(n_rows // CHUNK, CHUNK).astype(jnp.int32)
    out = _make_gather(n_rows)(ids, signs)
    return out.reshape(B, H, D)


# speedup vs baseline: 5.7952x; 1.0816x over previous
import functools

import jax
import jax.numpy as jnp
import numpy as np
from jax import lax
from jax.experimental import pallas as pl
from jax.experimental.pallas import tpu as pltpu
from jax.experimental.pallas import tpu_sc as plsc

D = 128
W = D // 4
L = 16

_info = plsc.get_sparse_core_info()
NC, NS = _info.num_cores, _info.num_subcores
NW = NC * NS

CHUNK = 128


def _pack_matrix(b_lo: int) -> np.ndarray:
    p = np.zeros((D, W), dtype=np.float32)
    for v in range(2):
        for b in (b_lo, b_lo + 1):
            for l in range(L):
                p[64 * v + 16 * b + l, 16 * v + l] = float(256 ** (b - b_lo))
    return p


def _pack_body(t_ref, plo_ref, phi_ref, o_ref):
    signs = (t_ref[...] >= 0).astype(jnp.float32)
    lo = jax.lax.dot(signs, plo_ref[...],
                     preferred_element_type=jnp.float32).astype(jnp.int32)
    hi = jax.lax.dot(signs, phi_ref[...],
                     preferred_element_type=jnp.float32).astype(jnp.int32)
    o_ref[...] = lo | (hi << 16)


def _packed_sign_table(table):
    V, d = table.shape
    blk = 800
    return pl.pallas_call(
        _pack_body,
        grid=(V // blk,),
        in_specs=[
            pl.BlockSpec((blk, d), lambda i: (i, 0)),
            pl.BlockSpec((d, W), lambda i: (0, 0)),
            pl.BlockSpec((d, W), lambda i: (0, 0)),
        ],
        out_specs=pl.BlockSpec((blk, W), lambda i: (i, 0)),
        out_shape=jax.ShapeDtypeStruct((V, W), jnp.int32),
    )(table, jnp.asarray(_pack_matrix(0)), jnp.asarray(_pack_matrix(2)))


def _make_gather(n_rows):
    per_w = n_rows // NW
    n_chunks = per_w // CHUNK
    assert n_chunks % 2 == 0
    mesh = plsc.VectorSubcoreMesh(core_axis_name="c", subcore_axis_name="s")

    @functools.partial(
        pl.kernel,
        mesh=mesh,
        out_type=jax.ShapeDtypeStruct((n_rows // CHUNK, CHUNK, D), jnp.float32),
        compiler_params=pltpu.CompilerParams(use_tc_tiling_on_sc=False),
        scratch_types=[
            pltpu.VMEM((n_chunks, CHUNK), jnp.int32),
            pltpu.VMEM((CHUNK, W), jnp.int32),
            pltpu.VMEM((CHUNK, W), jnp.int32),
            pltpu.VMEM((CHUNK, D), jnp.float32),
            pltpu.VMEM((CHUNK, D), jnp.float32),
            pltpu.SemaphoreType.DMA,
            pltpu.SemaphoreType.DMA,
            pltpu.SemaphoreType.DMA,
            pltpu.SemaphoreType.DMA,
        ],
    )
    def gather_k(ids_hbm, table_hbm, out_hbm, idx_v, w0, w1, o0, o1,
                 sg0, sg1, sw0, sw1):
        wbuf, obuf = (w0, w1), (o0, o1)
        sg, sw = (sg0, sg1), (sw0, sw1)
        wid = lax.axis_index("s") * NC + lax.axis_index("c")
        base = wid * n_chunks
        pltpu.sync_copy(ids_hbm.at[pl.ds(base, n_chunks)], idx_v)

        pltpu.async_copy(table_hbm.at[idx_v.at[0]], wbuf[0], sg[0])
        pltpu.async_copy(table_hbm.at[idx_v.at[1]], wbuf[1], sg[1])

        def decode(src, dst):
            def row(r, carry):
                for v in range(2):
                    words = src[r, pl.ds(16 * v, L)]
                    for b in range(4):
                        vals = ((words >> (8 * b)) & 1).astype(jnp.float32)
                        dst[r, pl.ds(64 * v + 16 * b, L)] = vals
                return carry
            lax.fori_loop(0, CHUNK, row, 0)

        def group(g, carry):
            for b in range(2):
                j = 2 * g + b
                pltpu.make_async_copy(
                    table_hbm.at[idx_v.at[j]], wbuf[b], sg[b]).wait()
                @pl.when(g >= 1)
                def _():
                    pltpu.make_async_copy(
                        obuf[b], out_hbm.at[base + j], sw[b]).wait()
                decode(wbuf[b], obuf[b])
                @pl.when(g < n_chunks // 2 - 1)
                def _():
                    pltpu.async_copy(
                        table_hbm.at[idx_v.at[j + 2]], wbuf[b], sg[b])
                pltpu.async_copy(obuf[b], out_hbm.at[base + j], sw[b])
            return carry

        lax.fori_loop(0, n_chunks // 2, group, 0)

        for b in range(2):
            pltpu.make_async_copy(
                obuf[b], out_hbm.at[base + n_chunks - 2 + b], sw[b]).wait()

    return gather_k


def kernel(input_ids, table):
    B, H = input_ids.shape
    n_rows = B * H
    packed = _packed_sign_table(table)
    ids = input_ids.reshape(n_rows // CHUNK, CHUNK).astype(jnp.int32)
    out = _make_gather(n_rows)(ids, packed)
    return out.reshape(B, H, D)


# speedup vs baseline: 6.9477x; 1.1989x over previous
import functools

import jax
import jax.numpy as jnp
from jax import lax
from jax.experimental import pallas as pl
from jax.experimental.pallas import tpu as pltpu
from jax.experimental.pallas import tpu_sc as plsc

D = 128
W = D // 4
L = 16

_info = plsc.get_sparse_core_info()
NC, NS = _info.num_cores, _info.num_subcores
NW = NC * NS

CHUNK = 128
PCHUNK = 160

_SC_PARAMS = pltpu.CompilerParams(use_tc_tiling_on_sc=False)


def _make_pack(V):
    n_chunks = -(-V // PCHUNK)
    assert V % PCHUNK == 0
    mesh = plsc.VectorSubcoreMesh(core_axis_name="c", subcore_axis_name="s")

    @functools.partial(
        pl.kernel,
        mesh=mesh,
        out_type=jax.ShapeDtypeStruct((V, W), jnp.int32),
        compiler_params=_SC_PARAMS,
        scratch_types=[
            pltpu.VMEM((PCHUNK, D), jnp.float32),
            pltpu.VMEM((PCHUNK, D), jnp.float32),
            pltpu.VMEM((PCHUNK, W), jnp.int32),
            pltpu.VMEM((PCHUNK, W), jnp.int32),
            pltpu.SemaphoreType.DMA,
            pltpu.SemaphoreType.DMA,
            pltpu.SemaphoreType.DMA,
            pltpu.SemaphoreType.DMA,
        ],
    )
    def pack_k(tab_hbm, out_hbm, t0, t1, p0, p1, sg0, sg1, sw0, sw1):
        tbuf, pbuf = (t0, t1), (p0, p1)
        sg, sw = (sg0, sg1), (sw0, sw1)
        wid = lax.axis_index("s") * NC + lax.axis_index("c")
        per_tile = -(-n_chunks // NW)

        def chunk_rows(i):
            c = jnp.minimum(wid + i * NW, n_chunks - 1)
            return c * PCHUNK

        pltpu.async_copy(tab_hbm.at[pl.ds(chunk_rows(0), PCHUNK)], tbuf[0],
                         sg[0])
        pltpu.async_copy(tab_hbm.at[pl.ds(chunk_rows(1), PCHUNK)], tbuf[1],
                         sg[1])

        def encode(src, dst):
            def row(r, carry):
                for v in range(2):
                    word = None
                    for b in range(4):
                        x = src[r, pl.ds(64 * v + 16 * b, L)]
                        s = jnp.where(x >= 0, jnp.int32(1 << (8 * b)),
                                      jnp.int32(0))
                        word = s if word is None else word | s
                    dst[r, pl.ds(16 * v, L)] = word
                return carry
            lax.fori_loop(0, PCHUNK, row, 0)

        def step(i, carry):
            for b in range(2):
                g = 2 * i + b
                rows = chunk_rows(g)
                pltpu.make_async_copy(
                    tab_hbm.at[pl.ds(rows, PCHUNK)], tbuf[b], sg[b]).wait()
                @pl.when(g >= 2)
                def _():
                    pltpu.make_async_copy(
                        pbuf[b], out_hbm.at[pl.ds(rows, PCHUNK)], sw[b]).wait()
                encode(tbuf[b], pbuf[b])
                @pl.when(g + 2 < per_tile)
                def _():
                    pltpu.async_copy(
                        tab_hbm.at[pl.ds(chunk_rows(g + 2), PCHUNK)],
                        tbuf[b], sg[b])
                pltpu.async_copy(
                    pbuf[b], out_hbm.at[pl.ds(rows, PCHUNK)], sw[b])
            return carry

        lax.fori_loop(0, per_tile // 2, step, 0)
        for b in range(2):
            pltpu.make_async_copy(
                pbuf[b], out_hbm.at[pl.ds(chunk_rows(per_tile - 2 + b),
                                          PCHUNK)], sw[b]).wait()

    return pack_k


def _make_gather(n_rows):
    per_w = n_rows // NW
    n_chunks = per_w // CHUNK
    assert n_chunks % 2 == 0
    mesh = plsc.VectorSubcoreMesh(core_axis_name="c", subcore_axis_name="s")

    @functools.partial(
        pl.kernel,
        mesh=mesh,
        out_type=jax.ShapeDtypeStruct((n_rows // CHUNK, CHUNK, D), jnp.float32),
        compiler_params=_SC_PARAMS,
        scratch_types=[
            pltpu.VMEM((n_chunks, CHUNK), jnp.int32),
            pltpu.VMEM((CHUNK, W), jnp.int32),
            pltpu.VMEM((CHUNK, W), jnp.int32),
            pltpu.VMEM((CHUNK, D), jnp.float32),
            pltpu.VMEM((CHUNK, D), jnp.float32),
            pltpu.SemaphoreType.DMA,
            pltpu.SemaphoreType.DMA,
            pltpu.SemaphoreType.DMA,
            pltpu.SemaphoreType.DMA,
        ],
    )
    def gather_k(ids_hbm, table_hbm, out_hbm, idx_v, w0, w1, o0, o1,
                 sg0, sg1, sw0, sw1):
        wbuf, obuf = (w0, w1), (o0, o1)
        sg, sw = (sg0, sg1), (sw0, sw1)
        wid = lax.axis_index("s") * NC + lax.axis_index("c")
        base = wid * n_chunks
        pltpu.sync_copy(ids_hbm.at[pl.ds(base, n_chunks)], idx_v)

        pltpu.async_copy(table_hbm.at[idx_v.at[0]], wbuf[0], sg[0])
        pltpu.async_copy(table_hbm.at[idx_v.at[1]], wbuf[1], sg[1])

        def decode(src, dst):
            def row(r, carry):
                for v in range(2):
                    words = src[r, pl.ds(16 * v, L)]
                    for b in range(4):
                        vals = ((words >> (8 * b)) & 1).astype(jnp.float32)
                        dst[r, pl.ds(64 * v + 16 * b, L)] = vals
                return carry
            lax.fori_loop(0, CHUNK, row, 0)

        def group(g, carry):
            for b in range(2):
                j = 2 * g + b
                pltpu.make_async_copy(
                    table_hbm.at[idx_v.at[j]], wbuf[b], sg[b]).wait()
                @pl.when(g >= 1)
                def _():
                    pltpu.make_async_copy(
                        obuf[b], out_hbm.at[base + j], sw[b]).wait()
                decode(wbuf[b], obuf[b])
                @pl.when(g < n_chunks // 2 - 1)
                def _():
                    pltpu.async_copy(
                        table_hbm.at[idx_v.at[j + 2]], wbuf[b], sg[b])
                pltpu.async_copy(obuf[b], out_hbm.at[base + j], sw[b])
            return carry

        lax.fori_loop(0, n_chunks // 2, group, 0)

        for b in range(2):
            pltpu.make_async_copy(
                obuf[b], out_hbm.at[base + n_chunks - 2 + b], sw[b]).wait()

    return gather_k


def kernel(input_ids, table):
    B, H = input_ids.shape
    V = table.shape[0]
    n_rows = B * H
    packed = _make_pack(V)(table)
    ids = input_ids.reshape(n_rows // CHUNK, CHUNK).astype(jnp.int32)
    out = _make_gather(n_rows)(ids, packed)
    return out.reshape(B, H, D)
